# baseline (speedup 1.0000x reference)
import jax
import jax.numpy as jnp
import numpy as np
from jax.experimental import pallas as pl

_SUPPORT_P = 0.01
_K2L = float(-np.log(_SUPPORT_P) * np.log2(np.e))

_D = 8
_NB = 64
_S = 1024
_C = 16


def _bump_block_kernel(x_ref, am_ref, bm_ref, mags_ref, y_ref):
    xT = x_ref[:, :].T
    acc8 = jnp.zeros((8, _S), jnp.float32)
    for c in range(0, _NB, _C):
        qn = jnp.zeros((_C, _S), jnp.float32)
        m = jnp.zeros((_C, _S), jnp.float32)
        for d in range(_D):
            z = xT[d : d + 1, :] * am_ref[c : c + _C, d : d + 1] \
                - bm_ref[c : c + _C, d : d + 1]
            z2 = z * z
            qn = qn - z2
            m = jnp.maximum(m, z2)
        e = jnp.exp2(qn)
        v = mags_ref[c : c + _C, :] * jnp.where(m < _K2L, e, 0.0)
        acc8 = acc8 + v[0:8, :] + v[8:16, :]
    y_ref[:, :] = jnp.sum(acc8, axis=0, keepdims=True)


@jax.jit
def kernel(x, ctrs, band_widths, mags):
    n = x.shape[0]
    npad = -(-n // _S) * _S
    xp = jnp.pad(x, ((0, npad - n), (0, 0)))

    sql = float(np.sqrt(np.log2(np.e)))
    am = sql / band_widths
    bm = sql * ctrs / band_widths
    mags2 = mags.reshape(_NB, 1)

    grid = (npad // _S,)
    y = pl.pallas_call(
        _bump_block_kernel,
        grid=grid,
        in_specs=[
            pl.BlockSpec((_S, _D), lambda i: (i, 0)),
            pl.BlockSpec((_NB, _D), lambda i: (0, 0)),
            pl.BlockSpec((_NB, _D), lambda i: (0, 0)),
            pl.BlockSpec((_NB, 1), lambda i: (0, 0)),
        ],
        out_specs=pl.BlockSpec((1, _S), lambda i: (0, i)),
        out_shape=jax.ShapeDtypeStruct((1, npad), jnp.float32),
    )(xp, am, bm, mags2)
    return y[0, :n]
